# baseline (speedup 1.0000x reference)
import dataclasses

import jax
import jax.numpy as jnp
from jax.experimental import pallas as pl
from jax.experimental.pallas import tpu as pltpu
from jax.experimental.pallas import tpu_sc as plsc

B, S, D = 16, 4096, 128
SBLK = 1024
ROWS_PER_SUBCORE = S // 16


def _mm_kernel(x_ref, m_ref, w_ref, o_ref):
    x = x_ref[...]
    m = m_ref[...]
    w = w_ref[...]
    y = jnp.dot(x.reshape(B * SBLK, D), w,
                preferred_element_type=jnp.float32).reshape(B, SBLK, D)
    o_ref[...] = y * m[:, :, None]


def _fin_kernel(g_ref, w_ref, o_ref):
    o_ref[...] = jnp.dot(g_ref[...], w_ref[...],
                         preferred_element_type=jnp.float32)


def _sc_compiler_params():
    cp = pltpu.CompilerParams()
    if "needs_layout_passes" in pltpu.CompilerParams.__dataclass_fields__:
        cp = dataclasses.replace(cp, needs_layout_passes=False)
    return cp


def _sc_book_kernel(maskT_hbm, x2d_hbm, ri_hbm, nv_hbm, g_hbm, part_hbm,
                    mbuf, pbuf, lbuf, idxbuf, ribuf, nvbuf, gbuf, sem):
    c = jax.lax.axis_index("c")
    s = jax.lax.axis_index("s")

    @pl.when(c == 0)
    def _reduce_slice():
        pltpu.async_copy(
            maskT_hbm.at[pl.ds(s * ROWS_PER_SUBCORE, ROWS_PER_SUBCORE)],
            mbuf, sem).wait()
        acc = jax.lax.fori_loop(
            0, ROWS_PER_SUBCORE, lambda i, a: a + mbuf[i],
            jnp.zeros((16,), jnp.float32))
        pbuf[...] = acc
        pltpu.async_copy(pbuf, part_hbm.at[s], sem).wait()

    plsc.subcore_barrier()

    @pl.when((c == 0) & (s == 0))
    def _finish():
        pltpu.async_copy(part_hbm, lbuf, sem).wait()
        lens_f = lbuf[0]
        for j in range(1, 16):
            lens_f = lens_f + lbuf[j]
        lens = lens_f.astype(jnp.int32)
        iota = jax.lax.iota(jnp.int32, 16)
        keys = lens * 16 + (15 - iota)
        keys_sorted, perm = plsc.sort_key_val(keys, iota, descending=True)
        _, ri = plsc.sort_key_val(perm, iota)
        lens_sorted = jax.lax.shift_right_logical(keys_sorted, 4)
        nv = plsc.all_reduce_population_count(lens >= 1)
        fidx = perm * S + jnp.maximum(lens_sorted - 1, 0)
        idxbuf[...] = fidx
        ribuf[...] = ri
        nvbuf[...] = nv
        pltpu.sync_copy(x2d_hbm.at[idxbuf], gbuf)
        pltpu.async_copy(ribuf, ri_hbm, sem).wait()
        pltpu.async_copy(nvbuf, nv_hbm, sem).wait()
        pltpu.async_copy(gbuf, g_hbm, sem).wait()


@jax.jit
def kernel(inputs, mask, W):
    maskT = mask.T
    x2d = inputs.reshape(B * S, D)

    sc_book = pl.kernel(
        _sc_book_kernel,
        out_type=[
            jax.ShapeDtypeStruct((16,), jnp.int32),
            jax.ShapeDtypeStruct((16,), jnp.int32),
            jax.ShapeDtypeStruct((16, D), jnp.float32),
            jax.ShapeDtypeStruct((16, 16), jnp.float32),
        ],
        mesh=plsc.VectorSubcoreMesh(core_axis_name="c", subcore_axis_name="s"),
        scratch_types=[
            pltpu.VMEM((ROWS_PER_SUBCORE, 16), jnp.float32),
            pltpu.VMEM((16,), jnp.float32),
            pltpu.VMEM((16, 16), jnp.float32),
            pltpu.VMEM((16,), jnp.int32),
            pltpu.VMEM((16,), jnp.int32),
            pltpu.VMEM((16,), jnp.int32),
            pltpu.VMEM((16, D), jnp.float32),
            pltpu.SemaphoreType.DMA,
        ],
        compiler_params=_sc_compiler_params(),
    )
    ri, nv, gathered, _ = sc_book(maskT, x2d)

    restored = pl.pallas_call(
        _mm_kernel,
        grid=(S // SBLK,),
        in_specs=[
            pl.BlockSpec((B, SBLK, D), lambda k: (0, k, 0)),
            pl.BlockSpec((B, SBLK), lambda k: (0, k)),
            pl.BlockSpec((D, D), lambda k: (0, 0)),
        ],
        out_specs=pl.BlockSpec((B, SBLK, D), lambda k: (0, k, 0)),
        out_shape=jax.ShapeDtypeStruct((B, S, D), jnp.float32),
    )(inputs, mask, W)

    fin = pl.pallas_call(
        _fin_kernel,
        out_shape=jax.ShapeDtypeStruct((B, D), jnp.float32),
    )(gathered, W)

    return (restored, fin[None, :, :], ri, nv[0])
